# baseline (speedup 1.0000x reference)
import functools

import jax
import jax.numpy as jnp
from jax import lax
from jax.experimental import pallas as pl
from jax.experimental.pallas import tpu as pltpu
from jax.experimental.pallas import tpu_sc as plsc

_LANES = 16
_IDX_CHUNK = 16384


def _make_sc_gather(n, num_spatial, heads):
    nn = n * n
    mesh = plsc.VectorSubcoreMesh(core_axis_name="c", subcore_axis_name="s")

    @functools.partial(
        pl.kernel,
        out_type=jax.ShapeDtypeStruct((heads, n, n), jnp.float32),
        mesh=mesh,
        compiler_params=pltpu.CompilerParams(needs_layout_passes=False),
        scratch_types=[
            pltpu.VMEM((num_spatial * heads,), jnp.float32),
            pltpu.VMEM((2, _IDX_CHUNK), jnp.int32),
            pltpu.VMEM((n, n), jnp.float32),
            pltpu.SemaphoreType.DMA,
            pltpu.SemaphoreType.DMA,
        ],
    )
    def sc_gather(spatial_hbm, table_hbm, out_hbm, table_v, idx_v,
                  out_v, sem, idx_sem):
        cid = lax.axis_index("c")
        sid = lax.axis_index("s")
        h = sid * 2 + cid

        n_chunks = nn // _IDX_CHUNK
        rows_per_chunk = _IDX_CHUNK // n

        def idx_fetch(c):
            return pltpu.async_copy(
                spatial_hbm.at[pl.ds(c * _IDX_CHUNK, _IDX_CHUNK)],
                idx_v.at[c % 2], idx_sem)

        idx_pending = idx_fetch(0)
        pltpu.sync_copy(table_hbm, table_v)

        pending = []
        for c in range(n_chunks):
            idx_pending.wait()
            if c + 1 < n_chunks:
                idx_pending = idx_fetch(c + 1)
            buf = c % 2

            def row_body(r, c=c, buf=buf):
                row = c * rows_per_chunk + r
                for u in range(n // _LANES):
                    iv = idx_v[buf, pl.ds(r * n + u * _LANES, _LANES)]
                    out_v[row, pl.ds(u * _LANES, _LANES)] = (
                        plsc.load_gather(table_v, [iv * heads + h]))

            plsc.parallel_loop(0, rows_per_chunk, unroll=4)(row_body)

            if len(pending) == 2:
                pending.pop(0).wait()
            pending.append(
                pltpu.async_copy(
                    out_v.at[pl.ds(c * rows_per_chunk, rows_per_chunk)],
                    out_hbm.at[h, pl.ds(c * rows_per_chunk, rows_per_chunk)],
                    sem))
        for cp in pending:
            cp.wait()

    return sc_gather


def _make_tc_broadcast(batch, n, heads):
    def body(bias_ref, out_ref, sem):
        copies = [pltpu.async_copy(bias_ref, out_ref.at[b], sem)
                  for b in range(batch)]
        for cp in copies:
            cp.wait()

    return pl.pallas_call(
        body,
        in_specs=[pl.BlockSpec(memory_space=pltpu.VMEM)],
        out_specs=pl.BlockSpec(memory_space=pl.ANY),
        out_shape=jax.ShapeDtypeStruct((batch, heads, n, n), jnp.float32),
        scratch_shapes=[pltpu.SemaphoreType.DMA],
    )


def kernel(x, spatial, table):
    batch = x.shape[0]
    n = spatial.shape[0]
    num_spatial, heads = table.shape
    sp_flat = spatial.reshape(-1).astype(jnp.int32)
    tab_flat = table.reshape(-1)
    bias = _make_sc_gather(n, num_spatial, heads)(sp_flat, tab_flat)
    return _make_tc_broadcast(batch, n, heads)(bias)


# speedup vs baseline: 1.1763x; 1.1763x over previous
import functools

import jax
import jax.numpy as jnp
from jax import lax
from jax.experimental import pallas as pl
from jax.experimental.pallas import tpu as pltpu
from jax.experimental.pallas import tpu_sc as plsc

_LANES = 16
_IDX_CHUNK = 16384


def _make_sc_gather(n, num_spatial, heads):
    nn = n * n
    mesh = plsc.VectorSubcoreMesh(core_axis_name="c", subcore_axis_name="s")

    @functools.partial(
        pl.kernel,
        out_type=jax.ShapeDtypeStruct((heads, n, n), jnp.float32),
        mesh=mesh,
        compiler_params=pltpu.CompilerParams(needs_layout_passes=False),
        scratch_types=[
            pltpu.VMEM((num_spatial * heads,), jnp.float32),
            pltpu.VMEM((num_spatial,), jnp.float32),
            pltpu.VMEM((2, _IDX_CHUNK), jnp.int32),
            pltpu.VMEM((n, n), jnp.float32),
            pltpu.SemaphoreType.DMA,
            pltpu.SemaphoreType.DMA,
        ],
    )
    def sc_gather(spatial_hbm, table_hbm, out_hbm, table_v, row_v, idx_v,
                  out_v, sem, idx_sem):
        cid = lax.axis_index("c")
        sid = lax.axis_index("s")
        h = sid * 2 + cid

        n_chunks = nn // _IDX_CHUNK
        rows_per_chunk = _IDX_CHUNK // n

        def idx_fetch(c):
            return pltpu.async_copy(
                spatial_hbm.at[pl.ds(c * _IDX_CHUNK, _IDX_CHUNK)],
                idx_v.at[c % 2], idx_sem)

        idx_pending = idx_fetch(0)
        pltpu.sync_copy(table_hbm, table_v)

        for i in range(num_spatial // _LANES):
            s_idx = lax.iota(jnp.int32, _LANES) + (i * _LANES)
            row_v[pl.ds(i * _LANES, _LANES)] = plsc.load_gather(
                table_v, [s_idx * heads + h])

        pending = []
        for c in range(n_chunks):
            idx_pending.wait()
            if c + 1 < n_chunks:
                idx_pending = idx_fetch(c + 1)
            buf = c % 2

            def row_body(r, c=c, buf=buf):
                row = c * rows_per_chunk + r
                for u in range(n // _LANES):
                    iv = idx_v[buf, pl.ds(r * n + u * _LANES, _LANES)]
                    out_v[row, pl.ds(u * _LANES, _LANES)] = (
                        plsc.load_gather(row_v, [iv]))

            plsc.parallel_loop(0, rows_per_chunk, unroll=4)(row_body)

            if len(pending) == 2:
                pending.pop(0).wait()
            pending.append(
                pltpu.async_copy(
                    out_v.at[pl.ds(c * rows_per_chunk, rows_per_chunk)],
                    out_hbm.at[h, pl.ds(c * rows_per_chunk, rows_per_chunk)],
                    sem))
        for cp in pending:
            cp.wait()

    return sc_gather


def _make_tc_broadcast(batch, n, heads):
    def body(bias_ref, out_ref, sem):
        copies = [pltpu.async_copy(bias_ref, out_ref.at[b], sem)
                  for b in range(batch)]
        for cp in copies:
            cp.wait()

    return pl.pallas_call(
        body,
        in_specs=[pl.BlockSpec(memory_space=pltpu.VMEM)],
        out_specs=pl.BlockSpec(memory_space=pl.ANY),
        out_shape=jax.ShapeDtypeStruct((batch, heads, n, n), jnp.float32),
        scratch_shapes=[pltpu.SemaphoreType.DMA],
    )


def kernel(x, spatial, table):
    batch = x.shape[0]
    n = spatial.shape[0]
    num_spatial, heads = table.shape
    sp_flat = spatial.reshape(-1).astype(jnp.int32)
    tab_flat = table.reshape(-1)
    bias = _make_sc_gather(n, num_spatial, heads)(sp_flat, tab_flat)
    return _make_tc_broadcast(batch, n, heads)(bias)
